# baseline (speedup 1.0000x reference)
import jax
import jax.numpy as jnp
from jax import lax
from jax.experimental import pallas as pl
from jax.experimental.pallas import tpu as pltpu
from jax.experimental.pallas import tpu_sc as plsc

_B = 16384
_V = 256
_NC = 2
_NS = 16
_NW = _NC * _NS
_BPW = _B // _NW
_L = 16
_GRAN = 16
_NROWS = _V * _V * _V // _GRAN
_CHUNK = 128
_NCHUNK = _BPW // _CHUNK
_NGRP = _BPW // _L


def _gather_body(x0_hbm, x1_hbm, x2_hbm, board_hbm, out_hbm,
                 ints, buf, sem_in, sems):
    wid = lax.axis_index("s") * _NC + lax.axis_index("c")
    base = wid * _BPW

    c0 = pltpu.async_copy(x0_hbm.at[pl.ds(base, _BPW)], ints.at[0], sem_in)
    c1 = pltpu.async_copy(x1_hbm.at[pl.ds(base, _BPW)], ints.at[1], sem_in)
    c2 = pltpu.async_copy(x2_hbm.at[pl.ds(base, _BPW)], ints.at[2], sem_in)
    c0.wait()
    c1.wait()
    c2.wait()

    gpc = _CHUNK // _L
    copies = []
    for j in range(_NCHUNK):
        for i in range(j * gpc, (j + 1) * gpc):
            s = pl.ds(i * _L, _L)
            x1v = ints.at[1][s]
            x2v = ints.at[2][s]
            p = ((ints.at[0][s] << 16) | ((x1v >> 3) << 11)
                 | ((x2v >> 7) << 10) | ((x1v & 7) << 7) | (x2v & 127))
            ints.at[1][s] = p >> 4
            ints.at[2][s] = p & 15
        c = pl.ds(j * _CHUNK, _CHUNK)
        copies.append(pltpu.async_copy(board_hbm.at[ints.at[1].at[c]],
                                       buf.at[c], sems.at[j]))

    lane = lax.iota(jnp.int32, _L)
    outs = []
    for j in range(_NCHUNK):
        copies[j].wait()
        for i in range(j * gpc, (j + 1) * gpc):
            s = pl.ds(i * _L, _L)
            ints.at[0][s] = plsc.bitcast(
                plsc.load_gather(buf, [lane + i * _L, ints.at[2][s]]),
                jnp.int32)
        c = pl.ds(j * _CHUNK, _CHUNK)
        outs.append(pltpu.async_copy(
            ints.at[0].at[c], out_hbm.at[pl.ds(base + j * _CHUNK, _CHUNK)],
            sem_in))
    for o in outs:
        o.wait()


@jax.jit
def _gather_sc(x0, x1, x2, board16):
    mesh = plsc.VectorSubcoreMesh(core_axis_name="c", subcore_axis_name="s")
    f = pl.kernel(
        _gather_body,
        out_type=jax.ShapeDtypeStruct((_B,), jnp.int32),
        mesh=mesh,
        compiler_params=pltpu.CompilerParams(
            needs_layout_passes=False, use_tc_tiling_on_sc=False),
        scratch_types=[
            pltpu.VMEM((3, _BPW), jnp.int32),
            pltpu.VMEM((_BPW, _GRAN), jnp.float32),
            pltpu.SemaphoreType.DMA,
            pltpu.SemaphoreType.DMA((_NCHUNK,)),
        ],
    )
    return f(x0, x1, x2, board16)


def kernel(x0, x1, x2, board):
    x0 = x0.astype(jnp.int32)
    x1 = x1.astype(jnp.int32)
    x2 = x2.astype(jnp.int32)
    board16 = (board.reshape(_V, 32, 8, 2, 128)
               .transpose(0, 1, 3, 2, 4)
               .reshape(_NROWS, _GRAN))
    out = lax.bitcast_convert_type(_gather_sc(x0, x1, x2, board16),
                                   jnp.float32)
    return out[:, None]


# speedup vs baseline: 1.0671x; 1.0671x over previous
import jax
import jax.numpy as jnp
from jax import lax
from jax.experimental import pallas as pl
from jax.experimental.pallas import tpu as pltpu
from jax.experimental.pallas import tpu_sc as plsc

_B = 16384
_V = 256
_NC = 2
_NS = 16
_NW = _NC * _NS
_BPW = _B // _NW
_L = 16
_GRAN = 16
_NROWS = _V * _V * _V // _GRAN
_CHUNK = 128
_NCHUNK = _BPW // _CHUNK
_NGRP = _BPW // _L


def _gather_body(x0_hbm, x1_hbm, x2_hbm, board_hbm, out_hbm,
                 ints, val_v, buf, sem_in, sems):
    wid = lax.axis_index("s") * _NC + lax.axis_index("c")
    base = wid * _BPW

    c0 = pltpu.async_copy(x0_hbm.at[pl.ds(base, _BPW)], ints.at[0], sem_in)
    c1 = pltpu.async_copy(x1_hbm.at[pl.ds(base, _BPW)], ints.at[1], sem_in)
    c2 = pltpu.async_copy(x2_hbm.at[pl.ds(base, _BPW)], ints.at[2], sem_in)
    c0.wait()
    c1.wait()
    c2.wait()

    gpc = _CHUNK // _L
    copies = []
    for j in range(_NCHUNK):
        for i in range(j * gpc, (j + 1) * gpc):
            s = pl.ds(i * _L, _L)
            x1v = ints.at[1][s]
            x2v = ints.at[2][s]
            p = ((ints.at[0][s] << 16) | ((x1v >> 3) << 11)
                 | ((x2v >> 7) << 10) | ((x1v & 7) << 7) | (x2v & 127))
            ints.at[1][s] = p >> 4
            ints.at[2][s] = p & 15
        c = pl.ds(j * _CHUNK, _CHUNK)
        copies.append(pltpu.async_copy(board_hbm.at[ints.at[1].at[c]],
                                       buf.at[c], sems.at[j]))

    lane = lax.iota(jnp.int32, _L)
    outs = []
    for j in range(_NCHUNK):
        copies[j].wait()
        for i in range(j * gpc, (j + 1) * gpc):
            s = pl.ds(i * _L, _L)
            val_v[s] = plsc.load_gather(buf, [lane + i * _L, ints.at[2][s]])
        c = pl.ds(j * _CHUNK, _CHUNK)
        outs.append(pltpu.async_copy(
            val_v.at[c], out_hbm.at[pl.ds(base + j * _CHUNK, _CHUNK)],
            sem_in))
    for o in outs:
        o.wait()


@jax.jit
def _gather_sc(x0, x1, x2, board16):
    mesh = plsc.VectorSubcoreMesh(core_axis_name="c", subcore_axis_name="s")
    f = pl.kernel(
        _gather_body,
        out_type=jax.ShapeDtypeStruct((_B,), jnp.float32),
        mesh=mesh,
        compiler_params=pltpu.CompilerParams(
            needs_layout_passes=False, use_tc_tiling_on_sc=False),
        scratch_types=[
            pltpu.VMEM((3, _BPW), jnp.int32),
            pltpu.VMEM((_BPW,), jnp.float32),
            pltpu.VMEM((_BPW, _GRAN), jnp.float32),
            pltpu.SemaphoreType.DMA,
            pltpu.SemaphoreType.DMA((_NCHUNK,)),
        ],
    )
    return f(x0, x1, x2, board16)


def kernel(x0, x1, x2, board):
    x0 = x0.astype(jnp.int32)
    x1 = x1.astype(jnp.int32)
    x2 = x2.astype(jnp.int32)
    board16 = (board.reshape(_V, 32, 8, 2, 128)
               .transpose(0, 1, 3, 2, 4)
               .reshape(_NROWS, _GRAN))
    out = _gather_sc(x0, x1, x2, board16)
    return out[:, None]
